# baseline (speedup 1.0000x reference)
import functools

import jax
import jax.numpy as jnp
from jax import lax
from jax.experimental import pallas as pl
from jax.experimental.pallas import tpu as pltpu
from jax.experimental.pallas import tpu_sc as plsc

_EMB = 32
_NW = 32
_CHUNK = 128


def _sc_gather(table, idx):
    b = idx.shape[0]
    d = table.shape[1]
    bpw = b // _NW
    k_chunks = bpw // _CHUNK
    idx2d = idx.reshape(b // _CHUNK, _CHUNK)
    mesh = plsc.VectorSubcoreMesh(core_axis_name="c", subcore_axis_name="s")

    @functools.partial(
        pl.kernel, mesh=mesh,
        out_type=jax.ShapeDtypeStruct((b, d), jnp.float32),
        scratch_types=[pltpu.VMEM((k_chunks, _CHUNK), jnp.int32),
                       pltpu.VMEM((bpw, d), jnp.float32),
                       pltpu.SemaphoreType.DMA],
        compiler_params=pltpu.CompilerParams(use_tc_tiling_on_sc=False),
    )
    def run(table_hbm, idx_hbm, out_hbm, idx_v, rows_v, sem):
        wid = lax.axis_index("s") * 2 + lax.axis_index("c")
        pltpu.sync_copy(idx_hbm.at[pl.ds(wid * k_chunks, k_chunks)], idx_v)
        copies = [pltpu.async_copy(table_hbm.at[idx_v.at[j]],
                                   rows_v.at[pl.ds(j * _CHUNK, _CHUNK)], sem)
                  for j in range(k_chunks)]
        for c in copies:
            c.wait()
        pltpu.sync_copy(rows_v, out_hbm.at[pl.ds(wid * bpw, bpw)])

    return run(table, idx2d)


def _ln(x, g, b):
    m = x.mean(-1, keepdims=True)
    var = ((x - m) ** 2).mean(-1, keepdims=True)
    return (x - m) / jnp.sqrt(var + 1e-6) * g + b


def _mlp(p, x):
    h = jnp.maximum(jnp.dot(x, p["w1"], preferred_element_type=jnp.float32) + p["b1"], 0.0)
    return jnp.dot(h, p["w2"], preferred_element_type=jnp.float32) + p["b2"]


def _attn_body(y, obs, qW, kW, vW, l1g, l1b, w1, b1, w2, b2, l2g, l2b, temp):
    q = jnp.dot(y, qW, preferred_element_type=jnp.float32)
    k = jnp.dot(y, kW, preferred_element_type=jnp.float32)
    vv = jnp.dot(y, vW, preferred_element_type=jnp.float32)
    ok = jnp.dot(obs, kW, preferred_element_type=jnp.float32)
    ov = jnp.dot(obs, vW, preferred_element_type=jnp.float32)
    zo = lax.dot_general(q, ok, (((1,), (1,)), ((), ())),
                         preferred_element_type=jnp.float32) / temp
    zs = jnp.sum(q * k, axis=-1, keepdims=True) / temp
    m = jnp.maximum(jnp.max(zo, axis=-1, keepdims=True), zs)
    es = jnp.exp(zs - m)
    eo = jnp.exp(zo - m)
    denom = es + jnp.sum(eo, axis=-1, keepdims=True)
    v_new = (es * vv + jnp.dot(eo, ov, preferred_element_type=jnp.float32)) / denom
    y = _ln(v_new + y, l1g, l1b)
    h = jnp.maximum(jnp.dot(y, w1, preferred_element_type=jnp.float32) + b1, 0.0)
    h = jnp.dot(h, w2, preferred_element_type=jnp.float32) + b2
    return _ln(h + y, l2g, l2b)


def _attn3_kernel(y_ref, obs_ref, qW, kW, vW, l1g, l1b, w1, b1, w2, b2, l2g, l2b, out_ref):
    y = y_ref[...]
    obs = obs_ref[...]
    temp = jnp.float32(float(_EMB) ** 0.5)
    for blk in range(3):
        y = _attn_body(y, obs, qW[blk], kW[blk], vW[blk], l1g[blk], l1b[blk],
                       w1[blk], b1[blk], w2[blk], b2[blk], l2g[blk], l2b[blk], temp)
    out_ref[...] = y


def _edge_mlp_kernel(with_fy, a_ref, b_ref, y_ref, w1y, b1y, w2y, b2y,
                     w1x, b1x, w2x, b2x, yout_ref, msg_ref):
    a = a_ref[...]
    b = b_ref[...]
    y = y_ref[...]
    if with_fy:
        zy = jnp.concatenate([b - a, b, a], axis=-1)
        h = jnp.maximum(jnp.dot(zy, w1y[...], preferred_element_type=jnp.float32) + b1y[...], 0.0)
        y = jnp.maximum(y, jnp.dot(h, w2y[...], preferred_element_type=jnp.float32) + b2y[...])
    yout_ref[...] = y
    zx = jnp.concatenate([a - b, a, b, y], axis=-1)
    h = jnp.maximum(jnp.dot(zx, w1x[...], preferred_element_type=jnp.float32) + b1x[...], 0.0)
    msg_ref[...] = jnp.dot(h, w2x[...], preferred_element_type=jnp.float32) + b2x[...]


def _edge_mlps(gx, y, pfy, pfx, with_fy):
    e = y.shape[0]
    blk = 8000
    grid = e // blk
    full = lambda s: pl.BlockSpec(s, lambda i: (0,) * len(s))
    wargs = [pfy["w1"], pfy["b1"], pfy["w2"], pfy["b2"],
             pfx["w1"], pfx["b1"], pfx["w2"], pfx["b2"]]
    espec = pl.BlockSpec((blk, _EMB), lambda i: (i, 0))
    return pl.pallas_call(
        functools.partial(_edge_mlp_kernel, with_fy),
        grid=(grid,),
        in_specs=[pl.BlockSpec((blk, _EMB), lambda i: (i, 0)),
                  pl.BlockSpec((blk, _EMB), lambda i: (i + grid, 0)),
                  espec] + [full(w.shape) for w in wargs],
        out_specs=(espec, espec),
        out_shape=(jax.ShapeDtypeStruct((e, _EMB), jnp.float32),
                   jax.ShapeDtypeStruct((e, _EMB), jnp.float32)),
    )(gx, gx, y, *wargs)


def _stack(blocks, *path):
    def get(b):
        for k in path:
            b = b[k]
        return b
    return jnp.stack([get(b) for b in blocks])


def _attn3(y, obs, blocks, block_rows):
    n = y.shape[0]
    assert n % block_rows == 0
    grid = n // block_rows
    ws = [_stack(blocks, "qW"), _stack(blocks, "kW"), _stack(blocks, "vW"),
          _stack(blocks, "ln1_g"), _stack(blocks, "ln1_b"),
          _stack(blocks, "ff", "w1"), _stack(blocks, "ff", "b1"),
          _stack(blocks, "ff", "w2"), _stack(blocks, "ff", "b2"),
          _stack(blocks, "ln2_g"), _stack(blocks, "ln2_b")]
    full = lambda s: pl.BlockSpec(s, lambda i: (0,) * len(s))
    in_specs = [pl.BlockSpec((block_rows, _EMB), lambda i: (i, 0)),
                full(obs.shape)] + [full(w.shape) for w in ws]
    return pl.pallas_call(
        _attn3_kernel,
        grid=(grid,),
        in_specs=in_specs,
        out_specs=pl.BlockSpec((block_rows, _EMB), lambda i: (i, 0)),
        out_shape=jax.ShapeDtypeStruct((n, _EMB), jnp.float32),
    )(y, obs, *ws)


_NPAD = 1024
_EPW = 2000
_GPT = 4096


def _sc_mpnn(x_pad, msg, dst, idx2d):
    mesh = plsc.VectorSubcoreMesh(core_axis_name="c", subcore_axis_name="s",
                                  num_cores=1)

    @functools.partial(
        pl.kernel, mesh=mesh,
        out_type=(jax.ShapeDtypeStruct((_NPAD, 32), jnp.float32),
                  jax.ShapeDtypeStruct((16 * _GPT, 32), jnp.float32),
                  jax.ShapeDtypeStruct((16, _NPAD, 32), jnp.float32)),
        scratch_types=[pltpu.VMEM((2048, 32), jnp.float32),
                       pltpu.VMEM((_EPW,), jnp.int32),
                       pltpu.VMEM((_NPAD, 32), jnp.float32),
                       pltpu.VMEM((64, 32), jnp.float32),
                       pltpu.VMEM((64, 32), jnp.float32),
                       pltpu.VMEM((32, 128), jnp.int32),
                       pltpu.SemaphoreType.DMA],
        compiler_params=pltpu.CompilerParams(use_tc_tiling_on_sc=False),
    )
    def run(x_hbm, msg_hbm, dst_hbm, idx_hbm, xout_hbm, gx_hbm, part_hbm,
            buf_v, dst_v, acc_v, comb_v, tmp_v, idx_v, sem):
        t = lax.axis_index("s")
        pltpu.sync_copy(x_hbm, acc_v)
        pltpu.sync_copy(msg_hbm.at[pl.ds(t * _EPW, _EPW)],
                        buf_v.at[pl.ds(0, _EPW)])
        pltpu.sync_copy(dst_hbm.at[pl.ds(t * _EPW, _EPW)], dst_v)

        def edge_body(g, carry):
            dvec = dst_v[pl.ds(g * 16, 16)]
            base = g * 16
            for l in range(16):
                d = dvec[l]
                for h in (0, 16):
                    m = buf_v[base + l, pl.ds(h, 16)]
                    a = acc_v[d, pl.ds(h, 16)]
                    acc_v[d, pl.ds(h, 16)] = jnp.maximum(a, m)
            return carry

        lax.fori_loop(0, _EPW // 16, edge_body, 0)
        pltpu.sync_copy(acc_v, part_hbm.at[t])
        plsc.subcore_barrier()
        pltpu.sync_copy(part_hbm.at[0, pl.ds(t * 64, 64)], comb_v)

        def comb_body(j, carry):
            pltpu.sync_copy(part_hbm.at[j, pl.ds(t * 64, 64)], tmp_v)

            def row_body(r, c2):
                for h in (0, 16):
                    comb_v[r, pl.ds(h, 16)] = jnp.maximum(
                        comb_v[r, pl.ds(h, 16)], tmp_v[r, pl.ds(h, 16)])
                return c2

            return lax.fori_loop(0, 64, row_body, carry)

        lax.fori_loop(1, 16, comb_body, 0)
        pltpu.sync_copy(comb_v, xout_hbm.at[pl.ds(t * 64, 64)])
        plsc.subcore_barrier()
        pltpu.sync_copy(idx_hbm.at[pl.ds(t * 32, 32)], idx_v)
        for h in range(2):
            copies = [pltpu.async_copy(xout_hbm.at[idx_v.at[h * 16 + j]],
                                       buf_v.at[pl.ds(j * 128, 128)], sem)
                      for j in range(16)]
            for cp in copies:
                cp.wait()
            pltpu.sync_copy(buf_v, gx_hbm.at[pl.ds(t * _GPT + h * 2048, 2048)])

    return run(x_pad, msg, dst, idx2d)


_TROWS = 31250
_EPT = 2048
_ZCH = 1536


def _sc_assemble(y_pad, idx2, n):
    mesh = plsc.VectorSubcoreMesh(core_axis_name="c", subcore_axis_name="s")
    zsrc = jnp.zeros((_ZCH, 32), jnp.float32)

    @functools.partial(
        pl.kernel, mesh=mesh,
        out_type=jax.ShapeDtypeStruct((n * n, 32), jnp.float32),
        scratch_types=[pltpu.VMEM((_ZCH, 32), jnp.float32),
                       pltpu.VMEM((_EPT, 32), jnp.float32),
                       pltpu.VMEM((16, 128), jnp.int32),
                       pltpu.SemaphoreType.DMA,
                       pltpu.SemaphoreType.DMA],
        compiler_params=pltpu.CompilerParams(use_tc_tiling_on_sc=False),
    )
    def run(y_hbm, idx_hbm, zsrc_hbm, out_hbm, zbuf, rows_v, idx_v, zsem, ssem):
        c = lax.axis_index("c")
        t = lax.axis_index("s")
        pltpu.sync_copy(zsrc_hbm, zbuf)
        base = (c * 16 + t) * _TROWS
        zcopies = [pltpu.async_copy(zbuf, out_hbm.at[pl.ds(base + k * _ZCH, _ZCH)], zsem)
                   for k in range(20)]
        zcopies.append(pltpu.async_copy(zbuf.at[pl.ds(0, 530)],
                                        out_hbm.at[pl.ds(base + 20 * _ZCH, 530)], zsem))
        pltpu.sync_copy(y_hbm.at[pl.ds(t * _EPT, _EPT)], rows_v)
        pltpu.sync_copy(idx_hbm.at[pl.ds(t * 16, 16)], idx_v)
        for cp in zcopies:
            cp.wait()
        plsc.subcore_barrier()
        scopies = [pltpu.async_copy(rows_v.at[pl.ds(j * 128, 128)],
                                    out_hbm.at[idx_v.at[j]], ssem)
                   for j in range(16)]
        for cp in scopies:
            cp.wait()

    return run(y_pad, idx2, zsrc)


def kernel(v, labels, obstacles, pos_enc, edge_index, loop, params):
    n = v.shape[0]
    vcat = jnp.concatenate([v, labels], axis=-1)
    goal_idx = jnp.argmin(jnp.abs(labels[:, 0] - 1.0))
    goal = vcat[goal_idx][None, :]
    gr = jnp.broadcast_to(goal, vcat.shape)
    x = _mlp(params["hx"], jnp.concatenate([vcat, gr, vcat - gr, (vcat - gr) ** 2], axis=-1))

    src = edge_index[0]
    dst = edge_index[1]
    e = src.shape[0]
    both = jnp.concatenate([src, dst]).astype(jnp.int32)
    padlen = (-both.shape[0]) % (_NW * _CHUNK)
    both_pad = jnp.concatenate(
        [both, jnp.arange(padlen, dtype=jnp.int32) % n])

    vcat16 = jnp.pad(vcat, ((0, 0), (0, 16 - vcat.shape[1])))
    g = _sc_gather(vcat16, both_pad)
    vi = g[:e, :vcat.shape[1]]
    vj = g[e:2 * e, :vcat.shape[1]]
    y = _mlp(params["hy"], jnp.concatenate([vj - vi, vj, vi], axis=-1))

    obs_node = _mlp(params["onc"], obstacles) + pos_enc
    obs_edge = _mlp(params["oec"], obstacles) + pos_enc

    x = _attn3(x, obs_node, params["na"], block_rows=1000)
    y = _attn3(y, obs_edge, params["ea"], block_rows=8000)

    dst32 = dst.astype(jnp.int32)
    idx2d = both_pad.reshape(512, 128)
    x_pad = jnp.concatenate([x, jnp.zeros((_NPAD - n, _EMB), jnp.float32)])
    gx0 = _sc_gather(x, both_pad)
    _, msg0 = _edge_mlps(gx0, y, params["fy"], params["fx"], with_fy=False)

    def body(_, carry):
        x_pad, y, msg = carry
        x_pad, gx, _ = _sc_mpnn(x_pad, msg, dst32, idx2d)
        y, msg = _edge_mlps(gx, y, params["fy"], params["fx"], with_fy=True)
        return (x_pad, y, msg)

    x_pad, y, _ = lax.fori_loop(0, loop, body, (x_pad, y, msg0))
    x = x_pad[:n]

    e_pad = 16 * _EPT
    y_pad = jnp.concatenate(
        [y, jnp.broadcast_to(y[0], (e_pad - e, _EMB))])
    flat = src.astype(jnp.int32) * n + dst.astype(jnp.int32)
    flat_p = jnp.concatenate(
        [flat, jnp.broadcast_to(flat[0], (e_pad - e,))])
    idx2 = flat_p.reshape(256, 128)
    out_rows = _sc_assemble(y_pad, idx2, n)
    edge_feat = out_rows.reshape(n, n, _EMB)
    return (edge_feat, x)


# speedup vs baseline: 1.0207x; 1.0207x over previous
import functools

import jax
import jax.numpy as jnp
from jax import lax
from jax.experimental import pallas as pl
from jax.experimental.pallas import tpu as pltpu
from jax.experimental.pallas import tpu_sc as plsc

_EMB = 32
_NW = 32
_CHUNK = 128


def _sc_gather(table, idx):
    b = idx.shape[0]
    d = table.shape[1]
    bpw = b // _NW
    k_chunks = bpw // _CHUNK
    idx2d = idx.reshape(b // _CHUNK, _CHUNK)
    mesh = plsc.VectorSubcoreMesh(core_axis_name="c", subcore_axis_name="s")

    @functools.partial(
        pl.kernel, mesh=mesh,
        out_type=jax.ShapeDtypeStruct((b, d), jnp.float32),
        scratch_types=[pltpu.VMEM((k_chunks, _CHUNK), jnp.int32),
                       pltpu.VMEM((bpw, d), jnp.float32),
                       pltpu.SemaphoreType.DMA],
        compiler_params=pltpu.CompilerParams(use_tc_tiling_on_sc=False),
    )
    def run(table_hbm, idx_hbm, out_hbm, idx_v, rows_v, sem):
        wid = lax.axis_index("s") * 2 + lax.axis_index("c")
        pltpu.sync_copy(idx_hbm.at[pl.ds(wid * k_chunks, k_chunks)], idx_v)
        copies = [pltpu.async_copy(table_hbm.at[idx_v.at[j]],
                                   rows_v.at[pl.ds(j * _CHUNK, _CHUNK)], sem)
                  for j in range(k_chunks)]
        for c in copies:
            c.wait()
        pltpu.sync_copy(rows_v, out_hbm.at[pl.ds(wid * bpw, bpw)])

    return run(table, idx2d)


def _ln(x, g, b):
    m = x.mean(-1, keepdims=True)
    var = ((x - m) ** 2).mean(-1, keepdims=True)
    return (x - m) / jnp.sqrt(var + 1e-6) * g + b


def _mlp(p, x):
    h = jnp.maximum(jnp.dot(x, p["w1"], preferred_element_type=jnp.float32) + p["b1"], 0.0)
    return jnp.dot(h, p["w2"], preferred_element_type=jnp.float32) + p["b2"]


def _attn_body(y, obs, qW, kW, vW, l1g, l1b, w1, b1, w2, b2, l2g, l2b, temp):
    q = jnp.dot(y, qW, preferred_element_type=jnp.float32)
    k = jnp.dot(y, kW, preferred_element_type=jnp.float32)
    vv = jnp.dot(y, vW, preferred_element_type=jnp.float32)
    ok = jnp.dot(obs, kW, preferred_element_type=jnp.float32)
    ov = jnp.dot(obs, vW, preferred_element_type=jnp.float32)
    zo = lax.dot_general(q, ok, (((1,), (1,)), ((), ())),
                         preferred_element_type=jnp.float32) / temp
    zs = jnp.sum(q * k, axis=-1, keepdims=True) / temp
    m = jnp.maximum(jnp.max(zo, axis=-1, keepdims=True), zs)
    es = jnp.exp(zs - m)
    eo = jnp.exp(zo - m)
    denom = es + jnp.sum(eo, axis=-1, keepdims=True)
    v_new = (es * vv + jnp.dot(eo, ov, preferred_element_type=jnp.float32)) / denom
    y = _ln(v_new + y, l1g, l1b)
    h = jnp.maximum(jnp.dot(y, w1, preferred_element_type=jnp.float32) + b1, 0.0)
    h = jnp.dot(h, w2, preferred_element_type=jnp.float32) + b2
    return _ln(h + y, l2g, l2b)


def _attn3_kernel(y_ref, obs_ref, qW, kW, vW, l1g, l1b, w1, b1, w2, b2, l2g, l2b, out_ref):
    y = y_ref[...]
    obs = obs_ref[...]
    temp = jnp.float32(float(_EMB) ** 0.5)
    for blk in range(3):
        y = _attn_body(y, obs, qW[blk], kW[blk], vW[blk], l1g[blk], l1b[blk],
                       w1[blk], b1[blk], w2[blk], b2[blk], l2g[blk], l2b[blk], temp)
    out_ref[...] = y


def _edge_mlp_kernel(with_fy, a_ref, b_ref, y_ref, w1y, b1y, w2y, b2y,
                     w1x, b1x, w2x, b2x, yout_ref, msg_ref):
    a = a_ref[...]
    b = b_ref[...]
    y = y_ref[...]
    if with_fy:
        zy = jnp.concatenate([b - a, b, a], axis=-1)
        h = jnp.maximum(jnp.dot(zy, w1y[...], preferred_element_type=jnp.float32) + b1y[...], 0.0)
        y = jnp.maximum(y, jnp.dot(h, w2y[...], preferred_element_type=jnp.float32) + b2y[...])
    yout_ref[...] = y
    zx = jnp.concatenate([a - b, a, b, y], axis=-1)
    h = jnp.maximum(jnp.dot(zx, w1x[...], preferred_element_type=jnp.float32) + b1x[...], 0.0)
    msg_ref[...] = jnp.dot(h, w2x[...], preferred_element_type=jnp.float32) + b2x[...]


def _edge_mlps(gx, y, pfy, pfx, with_fy):
    e = y.shape[0]
    blk = 4000
    grid = e // blk
    full = lambda s: pl.BlockSpec(s, lambda i: (0,) * len(s))
    wargs = [pfy["w1"], pfy["b1"], pfy["w2"], pfy["b2"],
             pfx["w1"], pfx["b1"], pfx["w2"], pfx["b2"]]
    espec = pl.BlockSpec((blk, _EMB), lambda i: (i, 0))
    return pl.pallas_call(
        functools.partial(_edge_mlp_kernel, with_fy),
        grid=(grid,),
        in_specs=[pl.BlockSpec((blk, _EMB), lambda i: (i, 0)),
                  pl.BlockSpec((blk, _EMB), lambda i: (i + grid, 0)),
                  espec] + [full(w.shape) for w in wargs],
        out_specs=(espec, espec),
        out_shape=(jax.ShapeDtypeStruct((e, _EMB), jnp.float32),
                   jax.ShapeDtypeStruct((e, _EMB), jnp.float32)),
    )(gx, gx, y, *wargs)


def _stack(blocks, *path):
    def get(b):
        for k in path:
            b = b[k]
        return b
    return jnp.stack([get(b) for b in blocks])


def _attn3(y, obs, blocks, block_rows):
    n = y.shape[0]
    assert n % block_rows == 0
    grid = n // block_rows
    ws = [_stack(blocks, "qW"), _stack(blocks, "kW"), _stack(blocks, "vW"),
          _stack(blocks, "ln1_g"), _stack(blocks, "ln1_b"),
          _stack(blocks, "ff", "w1"), _stack(blocks, "ff", "b1"),
          _stack(blocks, "ff", "w2"), _stack(blocks, "ff", "b2"),
          _stack(blocks, "ln2_g"), _stack(blocks, "ln2_b")]
    full = lambda s: pl.BlockSpec(s, lambda i: (0,) * len(s))
    in_specs = [pl.BlockSpec((block_rows, _EMB), lambda i: (i, 0)),
                full(obs.shape)] + [full(w.shape) for w in ws]
    return pl.pallas_call(
        _attn3_kernel,
        grid=(grid,),
        in_specs=in_specs,
        out_specs=pl.BlockSpec((block_rows, _EMB), lambda i: (i, 0)),
        out_shape=jax.ShapeDtypeStruct((n, _EMB), jnp.float32),
    )(y, obs, *ws)


_NPAD = 1024
_EPW = 2000
_GPT = 4096


def _sc_mpnn(x_pad, msg, dst, idx2d):
    mesh = plsc.VectorSubcoreMesh(core_axis_name="c", subcore_axis_name="s",
                                  num_cores=1)

    @functools.partial(
        pl.kernel, mesh=mesh,
        out_type=(jax.ShapeDtypeStruct((_NPAD, 32), jnp.float32),
                  jax.ShapeDtypeStruct((16 * _GPT, 32), jnp.float32),
                  jax.ShapeDtypeStruct((16, _NPAD, 32), jnp.float32)),
        scratch_types=[pltpu.VMEM((2048, 32), jnp.float32),
                       pltpu.VMEM((_EPW,), jnp.int32),
                       pltpu.VMEM((_NPAD, 32), jnp.float32),
                       pltpu.VMEM((64, 32), jnp.float32),
                       pltpu.VMEM((64, 32), jnp.float32),
                       pltpu.VMEM((32, 128), jnp.int32),
                       pltpu.SemaphoreType.DMA],
        compiler_params=pltpu.CompilerParams(use_tc_tiling_on_sc=False),
    )
    def run(x_hbm, msg_hbm, dst_hbm, idx_hbm, xout_hbm, gx_hbm, part_hbm,
            buf_v, dst_v, acc_v, comb_v, tmp_v, idx_v, sem):
        t = lax.axis_index("s")
        pltpu.sync_copy(x_hbm, acc_v)
        pltpu.sync_copy(msg_hbm.at[pl.ds(t * _EPW, _EPW)],
                        buf_v.at[pl.ds(0, _EPW)])
        pltpu.sync_copy(dst_hbm.at[pl.ds(t * _EPW, _EPW)], dst_v)

        def edge_body(g, carry):
            dvec = dst_v[pl.ds(g * 16, 16)]
            base = g * 16
            for l in range(16):
                d = dvec[l]
                for h in (0, 16):
                    m = buf_v[base + l, pl.ds(h, 16)]
                    a = acc_v[d, pl.ds(h, 16)]
                    acc_v[d, pl.ds(h, 16)] = jnp.maximum(a, m)
            return carry

        lax.fori_loop(0, _EPW // 16, edge_body, 0)
        pltpu.sync_copy(acc_v, part_hbm.at[t])
        plsc.subcore_barrier()
        pltpu.sync_copy(part_hbm.at[0, pl.ds(t * 64, 64)], comb_v)

        def comb_body(j, carry):
            pltpu.sync_copy(part_hbm.at[j, pl.ds(t * 64, 64)], tmp_v)

            def row_body(r, c2):
                for h in (0, 16):
                    comb_v[r, pl.ds(h, 16)] = jnp.maximum(
                        comb_v[r, pl.ds(h, 16)], tmp_v[r, pl.ds(h, 16)])
                return c2

            return lax.fori_loop(0, 64, row_body, carry)

        lax.fori_loop(1, 16, comb_body, 0)
        pltpu.sync_copy(comb_v, xout_hbm.at[pl.ds(t * 64, 64)])
        plsc.subcore_barrier()
        pltpu.sync_copy(idx_hbm.at[pl.ds(t * 32, 32)], idx_v)
        for h in range(2):
            copies = [pltpu.async_copy(xout_hbm.at[idx_v.at[h * 16 + j]],
                                       buf_v.at[pl.ds(j * 128, 128)], sem)
                      for j in range(16)]
            for cp in copies:
                cp.wait()
            pltpu.sync_copy(buf_v, gx_hbm.at[pl.ds(t * _GPT + h * 2048, 2048)])

    return run(x_pad, msg, dst, idx2d)


_TROWS = 31250
_EPT = 2048
_ZCH = 1536


def _sc_assemble(y_pad, idx2, n):
    mesh = plsc.VectorSubcoreMesh(core_axis_name="c", subcore_axis_name="s")
    zsrc = jnp.zeros((_ZCH, 32), jnp.float32)

    @functools.partial(
        pl.kernel, mesh=mesh,
        out_type=jax.ShapeDtypeStruct((n * n, 32), jnp.float32),
        scratch_types=[pltpu.VMEM((_ZCH, 32), jnp.float32),
                       pltpu.VMEM((_EPT, 32), jnp.float32),
                       pltpu.VMEM((16, 128), jnp.int32),
                       pltpu.SemaphoreType.DMA,
                       pltpu.SemaphoreType.DMA],
        compiler_params=pltpu.CompilerParams(use_tc_tiling_on_sc=False),
    )
    def run(y_hbm, idx_hbm, zsrc_hbm, out_hbm, zbuf, rows_v, idx_v, zsem, ssem):
        c = lax.axis_index("c")
        t = lax.axis_index("s")
        pltpu.sync_copy(zsrc_hbm, zbuf)
        base = (c * 16 + t) * _TROWS
        zcopies = [pltpu.async_copy(zbuf, out_hbm.at[pl.ds(base + k * _ZCH, _ZCH)], zsem)
                   for k in range(20)]
        zcopies.append(pltpu.async_copy(zbuf.at[pl.ds(0, 530)],
                                        out_hbm.at[pl.ds(base + 20 * _ZCH, 530)], zsem))
        pltpu.sync_copy(y_hbm.at[pl.ds(t * _EPT, _EPT)], rows_v)
        pltpu.sync_copy(idx_hbm.at[pl.ds(t * 16, 16)], idx_v)
        for cp in zcopies:
            cp.wait()
        plsc.subcore_barrier()
        scopies = [pltpu.async_copy(rows_v.at[pl.ds(j * 128, 128)],
                                    out_hbm.at[idx_v.at[j]], ssem)
                   for j in range(16)]
        for cp in scopies:
            cp.wait()

    return run(y_pad, idx2, zsrc)


def kernel(v, labels, obstacles, pos_enc, edge_index, loop, params):
    n = v.shape[0]
    vcat = jnp.concatenate([v, labels], axis=-1)
    goal_idx = jnp.argmin(jnp.abs(labels[:, 0] - 1.0))
    goal = vcat[goal_idx][None, :]
    gr = jnp.broadcast_to(goal, vcat.shape)
    x = _mlp(params["hx"], jnp.concatenate([vcat, gr, vcat - gr, (vcat - gr) ** 2], axis=-1))

    src = edge_index[0]
    dst = edge_index[1]
    e = src.shape[0]
    both = jnp.concatenate([src, dst]).astype(jnp.int32)
    padlen = (-both.shape[0]) % (_NW * _CHUNK)
    both_pad = jnp.concatenate(
        [both, jnp.arange(padlen, dtype=jnp.int32) % n])

    vcat16 = jnp.pad(vcat, ((0, 0), (0, 16 - vcat.shape[1])))
    g = _sc_gather(vcat16, both_pad)
    vi = g[:e, :vcat.shape[1]]
    vj = g[e:2 * e, :vcat.shape[1]]
    y = _mlp(params["hy"], jnp.concatenate([vj - vi, vj, vi], axis=-1))

    obs_node = _mlp(params["onc"], obstacles) + pos_enc
    obs_edge = _mlp(params["oec"], obstacles) + pos_enc

    x = _attn3(x, obs_node, params["na"], block_rows=1000)
    y = _attn3(y, obs_edge, params["ea"], block_rows=4000)

    dst32 = dst.astype(jnp.int32)
    idx2d = both_pad.reshape(512, 128)
    x_pad = jnp.concatenate([x, jnp.zeros((_NPAD - n, _EMB), jnp.float32)])
    gx0 = _sc_gather(x, both_pad)
    _, msg0 = _edge_mlps(gx0, y, params["fy"], params["fx"], with_fy=False)

    def body(_, carry):
        x_pad, y, msg = carry
        x_pad, gx, _ = _sc_mpnn(x_pad, msg, dst32, idx2d)
        y, msg = _edge_mlps(gx, y, params["fy"], params["fx"], with_fy=True)
        return (x_pad, y, msg)

    x_pad, y, _ = lax.fori_loop(0, loop, body, (x_pad, y, msg0))
    x = x_pad[:n]

    e_pad = 16 * _EPT
    y_pad = jnp.concatenate(
        [y, jnp.broadcast_to(y[0], (e_pad - e, _EMB))])
    flat = src.astype(jnp.int32) * n + dst.astype(jnp.int32)
    flat_p = jnp.concatenate(
        [flat, jnp.broadcast_to(flat[0], (e_pad - e,))])
    idx2 = flat_p.reshape(256, 128)
    out_rows = _sc_assemble(y_pad, idx2, n)
    edge_feat = out_rows.reshape(n, n, _EMB)
    return (edge_feat, x)


# speedup vs baseline: 1.0505x; 1.0291x over previous
import functools

import jax
import jax.numpy as jnp
from jax import lax
from jax.experimental import pallas as pl
from jax.experimental.pallas import tpu as pltpu
from jax.experimental.pallas import tpu_sc as plsc

_EMB = 32
_NW = 32
_CHUNK = 128


def _sc_gather(table, idx):
    b = idx.shape[0]
    d = table.shape[1]
    bpw = b // _NW
    k_chunks = bpw // _CHUNK
    idx2d = idx.reshape(b // _CHUNK, _CHUNK)
    mesh = plsc.VectorSubcoreMesh(core_axis_name="c", subcore_axis_name="s")

    @functools.partial(
        pl.kernel, mesh=mesh,
        out_type=jax.ShapeDtypeStruct((b, d), jnp.float32),
        scratch_types=[pltpu.VMEM((k_chunks, _CHUNK), jnp.int32),
                       pltpu.VMEM((bpw, d), jnp.float32),
                       pltpu.SemaphoreType.DMA],
        compiler_params=pltpu.CompilerParams(use_tc_tiling_on_sc=False),
    )
    def run(table_hbm, idx_hbm, out_hbm, idx_v, rows_v, sem):
        wid = lax.axis_index("s") * 2 + lax.axis_index("c")
        pltpu.sync_copy(idx_hbm.at[pl.ds(wid * k_chunks, k_chunks)], idx_v)
        copies = [pltpu.async_copy(table_hbm.at[idx_v.at[j]],
                                   rows_v.at[pl.ds(j * _CHUNK, _CHUNK)], sem)
                  for j in range(k_chunks)]
        for c in copies:
            c.wait()
        pltpu.sync_copy(rows_v, out_hbm.at[pl.ds(wid * bpw, bpw)])

    return run(table, idx2d)


def _ln(x, g, b):
    m = x.mean(-1, keepdims=True)
    var = ((x - m) ** 2).mean(-1, keepdims=True)
    return (x - m) / jnp.sqrt(var + 1e-6) * g + b


def _mlp(p, x):
    h = jnp.maximum(jnp.dot(x, p["w1"], preferred_element_type=jnp.float32) + p["b1"], 0.0)
    return jnp.dot(h, p["w2"], preferred_element_type=jnp.float32) + p["b2"]


def _attn_body(y, obs, qW, kW, vW, l1g, l1b, w1, b1, w2, b2, l2g, l2b, temp):
    q = jnp.dot(y, qW, preferred_element_type=jnp.float32)
    k = jnp.dot(y, kW, preferred_element_type=jnp.float32)
    vv = jnp.dot(y, vW, preferred_element_type=jnp.float32)
    ok = jnp.dot(obs, kW, preferred_element_type=jnp.float32)
    ov = jnp.dot(obs, vW, preferred_element_type=jnp.float32)
    zo = lax.dot_general(q, ok, (((1,), (1,)), ((), ())),
                         preferred_element_type=jnp.float32) / temp
    zs = jnp.sum(q * k, axis=-1, keepdims=True) / temp
    m = jnp.maximum(jnp.max(zo, axis=-1, keepdims=True), zs)
    es = jnp.exp(zs - m)
    eo = jnp.exp(zo - m)
    denom = es + jnp.sum(eo, axis=-1, keepdims=True)
    v_new = (es * vv + jnp.dot(eo, ov, preferred_element_type=jnp.float32)) / denom
    y = _ln(v_new + y, l1g, l1b)
    h = jnp.maximum(jnp.dot(y, w1, preferred_element_type=jnp.float32) + b1, 0.0)
    h = jnp.dot(h, w2, preferred_element_type=jnp.float32) + b2
    return _ln(h + y, l2g, l2b)


def _attn3_kernel(y_ref, obs_ref, qW, kW, vW, l1g, l1b, w1, b1, w2, b2, l2g, l2b, out_ref):
    y = y_ref[...]
    obs = obs_ref[...]
    temp = jnp.float32(float(_EMB) ** 0.5)
    for blk in range(3):
        y = _attn_body(y, obs, qW[blk], kW[blk], vW[blk], l1g[blk], l1b[blk],
                       w1[blk], b1[blk], w2[blk], b2[blk], l2g[blk], l2b[blk], temp)
    out_ref[...] = y


def _hy_attn3_kernel(d_in, vi_ref, vj_ref, obs_ref, w1h, b1h, w2h, b2h,
                     qW, kW, vW, l1g, l1b, w1, b1, w2, b2, l2g, l2b, out_ref):
    vi = vi_ref[...][:, :d_in]
    vj = vj_ref[...][:, :d_in]
    z = jnp.concatenate([vj - vi, vj, vi], axis=-1)
    h = jnp.maximum(jnp.dot(z, w1h[...], preferred_element_type=jnp.float32) + b1h[...], 0.0)
    y = jnp.dot(h, w2h[...], preferred_element_type=jnp.float32) + b2h[...]
    obs = obs_ref[...]
    temp = jnp.float32(float(_EMB) ** 0.5)
    for blk in range(3):
        y = _attn_body(y, obs, qW[blk], kW[blk], vW[blk], l1g[blk], l1b[blk],
                       w1[blk], b1[blk], w2[blk], b2[blk], l2g[blk], l2b[blk], temp)
    out_ref[...] = y


def _hy_attn3(g, e, d_in, obs, phy, blocks):
    blk = 4000
    grid = e // blk
    ws = [phy["w1"], phy["b1"], phy["w2"], phy["b2"],
          _stack(blocks, "qW"), _stack(blocks, "kW"), _stack(blocks, "vW"),
          _stack(blocks, "ln1_g"), _stack(blocks, "ln1_b"),
          _stack(blocks, "ff", "w1"), _stack(blocks, "ff", "b1"),
          _stack(blocks, "ff", "w2"), _stack(blocks, "ff", "b2"),
          _stack(blocks, "ln2_g"), _stack(blocks, "ln2_b")]
    full = lambda s: pl.BlockSpec(s, lambda i: (0,) * len(s))
    gspec = lambda off: pl.BlockSpec((blk, g.shape[1]), lambda i, off=off: (i + off, 0))
    return pl.pallas_call(
        functools.partial(_hy_attn3_kernel, d_in),
        grid=(grid,),
        in_specs=[gspec(0), gspec(grid), full(obs.shape)] + [full(w.shape) for w in ws],
        out_specs=pl.BlockSpec((blk, _EMB), lambda i: (i, 0)),
        out_shape=jax.ShapeDtypeStruct((e, _EMB), jnp.float32),
    )(g, g, obs, *ws)


def _edge_mlp_kernel(with_fy, a_ref, b_ref, y_ref, w1y, b1y, w2y, b2y,
                     w1x, b1x, w2x, b2x, yout_ref, msg_ref):
    a = a_ref[...]
    b = b_ref[...]
    y = y_ref[...]
    if with_fy:
        zy = jnp.concatenate([b - a, b, a], axis=-1)
        h = jnp.maximum(jnp.dot(zy, w1y[...], preferred_element_type=jnp.float32) + b1y[...], 0.0)
        y = jnp.maximum(y, jnp.dot(h, w2y[...], preferred_element_type=jnp.float32) + b2y[...])
    yout_ref[...] = y
    zx = jnp.concatenate([a - b, a, b, y], axis=-1)
    h = jnp.maximum(jnp.dot(zx, w1x[...], preferred_element_type=jnp.float32) + b1x[...], 0.0)
    msg_ref[...] = jnp.dot(h, w2x[...], preferred_element_type=jnp.float32) + b2x[...]


def _edge_mlps(gx, y, pfy, pfx, with_fy):
    e = y.shape[0]
    blk = 4000
    grid = e // blk
    full = lambda s: pl.BlockSpec(s, lambda i: (0,) * len(s))
    wargs = [pfy["w1"], pfy["b1"], pfy["w2"], pfy["b2"],
             pfx["w1"], pfx["b1"], pfx["w2"], pfx["b2"]]
    espec = pl.BlockSpec((blk, _EMB), lambda i: (i, 0))
    return pl.pallas_call(
        functools.partial(_edge_mlp_kernel, with_fy),
        grid=(grid,),
        in_specs=[pl.BlockSpec((blk, _EMB), lambda i: (i, 0)),
                  pl.BlockSpec((blk, _EMB), lambda i: (i + grid, 0)),
                  espec] + [full(w.shape) for w in wargs],
        out_specs=(espec, espec),
        out_shape=(jax.ShapeDtypeStruct((e, _EMB), jnp.float32),
                   jax.ShapeDtypeStruct((e, _EMB), jnp.float32)),
    )(gx, gx, y, *wargs)


def _stack(blocks, *path):
    def get(b):
        for k in path:
            b = b[k]
        return b
    return jnp.stack([get(b) for b in blocks])


def _attn3(y, obs, blocks, block_rows):
    n = y.shape[0]
    assert n % block_rows == 0
    grid = n // block_rows
    ws = [_stack(blocks, "qW"), _stack(blocks, "kW"), _stack(blocks, "vW"),
          _stack(blocks, "ln1_g"), _stack(blocks, "ln1_b"),
          _stack(blocks, "ff", "w1"), _stack(blocks, "ff", "b1"),
          _stack(blocks, "ff", "w2"), _stack(blocks, "ff", "b2"),
          _stack(blocks, "ln2_g"), _stack(blocks, "ln2_b")]
    full = lambda s: pl.BlockSpec(s, lambda i: (0,) * len(s))
    in_specs = [pl.BlockSpec((block_rows, _EMB), lambda i: (i, 0)),
                full(obs.shape)] + [full(w.shape) for w in ws]
    return pl.pallas_call(
        _attn3_kernel,
        grid=(grid,),
        in_specs=in_specs,
        out_specs=pl.BlockSpec((block_rows, _EMB), lambda i: (i, 0)),
        out_shape=jax.ShapeDtypeStruct((n, _EMB), jnp.float32),
    )(y, obs, *ws)


_NPAD = 1024
_EPW = 2000
_GPT = 4096


def _sc_mpnn(x_pad, msg, dst, idx2d):
    mesh = plsc.VectorSubcoreMesh(core_axis_name="c", subcore_axis_name="s",
                                  num_cores=1)

    @functools.partial(
        pl.kernel, mesh=mesh,
        out_type=(jax.ShapeDtypeStruct((_NPAD, 32), jnp.float32),
                  jax.ShapeDtypeStruct((16 * _GPT, 32), jnp.float32),
                  jax.ShapeDtypeStruct((16, _NPAD, 32), jnp.float32)),
        scratch_types=[pltpu.VMEM((2048, 32), jnp.float32),
                       pltpu.VMEM((_EPW,), jnp.int32),
                       pltpu.VMEM((_NPAD, 32), jnp.float32),
                       pltpu.VMEM((64, 32), jnp.float32),
                       pltpu.VMEM((64, 32), jnp.float32),
                       pltpu.VMEM((32, 128), jnp.int32),
                       pltpu.SemaphoreType.DMA],
        compiler_params=pltpu.CompilerParams(use_tc_tiling_on_sc=False),
    )
    def run(x_hbm, msg_hbm, dst_hbm, idx_hbm, xout_hbm, gx_hbm, part_hbm,
            buf_v, dst_v, acc_v, comb_v, tmp_v, idx_v, sem):
        t = lax.axis_index("s")
        pltpu.sync_copy(x_hbm, acc_v)
        pltpu.sync_copy(msg_hbm.at[pl.ds(t * _EPW, _EPW)],
                        buf_v.at[pl.ds(0, _EPW)])
        pltpu.sync_copy(dst_hbm.at[pl.ds(t * _EPW, _EPW)], dst_v)

        def edge_body(g, carry):
            dvec = dst_v[pl.ds(g * 16, 16)]
            base = g * 16
            for l in range(16):
                d = dvec[l]
                for h in (0, 16):
                    m = buf_v[base + l, pl.ds(h, 16)]
                    a = acc_v[d, pl.ds(h, 16)]
                    acc_v[d, pl.ds(h, 16)] = jnp.maximum(a, m)
            return carry

        lax.fori_loop(0, _EPW // 16, edge_body, 0)
        pltpu.sync_copy(acc_v, part_hbm.at[t])
        plsc.subcore_barrier()
        pltpu.sync_copy(part_hbm.at[0, pl.ds(t * 64, 64)], comb_v)

        def comb_body(j, carry):
            pltpu.sync_copy(part_hbm.at[j, pl.ds(t * 64, 64)], tmp_v)

            def row_body(r, c2):
                for h in (0, 16):
                    comb_v[r, pl.ds(h, 16)] = jnp.maximum(
                        comb_v[r, pl.ds(h, 16)], tmp_v[r, pl.ds(h, 16)])
                return c2

            return lax.fori_loop(0, 64, row_body, carry)

        lax.fori_loop(1, 16, comb_body, 0)
        pltpu.sync_copy(comb_v, xout_hbm.at[pl.ds(t * 64, 64)])
        plsc.subcore_barrier()
        pltpu.sync_copy(idx_hbm.at[pl.ds(t * 32, 32)], idx_v)
        for h in range(2):
            copies = [pltpu.async_copy(xout_hbm.at[idx_v.at[h * 16 + j]],
                                       buf_v.at[pl.ds(j * 128, 128)], sem)
                      for j in range(16)]
            for cp in copies:
                cp.wait()
            pltpu.sync_copy(buf_v, gx_hbm.at[pl.ds(t * _GPT + h * 2048, 2048)])

    return run(x_pad, msg, dst, idx2d)


_TROWS = 31250
_EPT = 2048
_ZCH = 1536


def _sc_assemble(y_pad, idx2, n):
    mesh = plsc.VectorSubcoreMesh(core_axis_name="c", subcore_axis_name="s")
    zsrc = jnp.zeros((_ZCH, 32), jnp.float32)

    @functools.partial(
        pl.kernel, mesh=mesh,
        out_type=jax.ShapeDtypeStruct((n * n, 32), jnp.float32),
        scratch_types=[pltpu.VMEM((_ZCH, 32), jnp.float32),
                       pltpu.VMEM((_EPT, 32), jnp.float32),
                       pltpu.VMEM((16, 128), jnp.int32),
                       pltpu.SemaphoreType.DMA,
                       pltpu.SemaphoreType.DMA],
        compiler_params=pltpu.CompilerParams(use_tc_tiling_on_sc=False),
    )
    def run(y_hbm, idx_hbm, zsrc_hbm, out_hbm, zbuf, rows_v, idx_v, zsem, ssem):
        c = lax.axis_index("c")
        t = lax.axis_index("s")
        pltpu.sync_copy(zsrc_hbm, zbuf)
        base = (c * 16 + t) * _TROWS
        zcopies = [pltpu.async_copy(zbuf, out_hbm.at[pl.ds(base + k * _ZCH, _ZCH)], zsem)
                   for k in range(20)]
        zcopies.append(pltpu.async_copy(zbuf.at[pl.ds(0, 530)],
                                        out_hbm.at[pl.ds(base + 20 * _ZCH, 530)], zsem))
        pltpu.sync_copy(y_hbm.at[pl.ds(t * _EPT, _EPT)], rows_v)
        pltpu.sync_copy(idx_hbm.at[pl.ds(t * 16, 16)], idx_v)
        for cp in zcopies:
            cp.wait()
        plsc.subcore_barrier()
        scopies = [pltpu.async_copy(rows_v.at[pl.ds(j * 128, 128)],
                                    out_hbm.at[idx_v.at[j]], ssem)
                   for j in range(16)]
        for cp in scopies:
            cp.wait()

    return run(y_pad, idx2, zsrc)


def kernel(v, labels, obstacles, pos_enc, edge_index, loop, params):
    n = v.shape[0]
    vcat = jnp.concatenate([v, labels], axis=-1)
    goal_idx = jnp.argmin(jnp.abs(labels[:, 0] - 1.0))
    goal = vcat[goal_idx][None, :]
    gr = jnp.broadcast_to(goal, vcat.shape)
    x = _mlp(params["hx"], jnp.concatenate([vcat, gr, vcat - gr, (vcat - gr) ** 2], axis=-1))

    src = edge_index[0]
    dst = edge_index[1]
    e = src.shape[0]
    both = jnp.concatenate([src, dst]).astype(jnp.int32)
    padlen = (-both.shape[0]) % (_NW * _CHUNK)
    both_pad = jnp.concatenate(
        [both, jnp.arange(padlen, dtype=jnp.int32) % n])

    vcat16 = jnp.pad(vcat, ((0, 0), (0, 16 - vcat.shape[1])))
    g = _sc_gather(vcat16, both_pad)

    obs_node = _mlp(params["onc"], obstacles) + pos_enc
    obs_edge = _mlp(params["oec"], obstacles) + pos_enc

    x = _attn3(x, obs_node, params["na"], block_rows=1000)
    y = _hy_attn3(g, e, vcat.shape[1], obs_edge, params["hy"], params["ea"])

    dst32 = dst.astype(jnp.int32)
    idx2d = both_pad.reshape(512, 128)
    x_pad = jnp.concatenate([x, jnp.zeros((_NPAD - n, _EMB), jnp.float32)])
    gx0 = _sc_gather(x, both_pad)
    _, msg0 = _edge_mlps(gx0, y, params["fy"], params["fx"], with_fy=False)

    def body(_, carry):
        x_pad, y, msg = carry
        x_pad, gx, _ = _sc_mpnn(x_pad, msg, dst32, idx2d)
        y, msg = _edge_mlps(gx, y, params["fy"], params["fx"], with_fy=True)
        return (x_pad, y, msg)

    x_pad, y, _ = lax.fori_loop(0, loop, body, (x_pad, y, msg0))
    x = x_pad[:n]

    e_pad = 16 * _EPT
    y_pad = jnp.concatenate(
        [y, jnp.broadcast_to(y[0], (e_pad - e, _EMB))])
    flat = src.astype(jnp.int32) * n + dst.astype(jnp.int32)
    flat_p = jnp.concatenate(
        [flat, jnp.broadcast_to(flat[0], (e_pad - e,))])
    idx2 = flat_p.reshape(256, 128)
    out_rows = _sc_assemble(y_pad, idx2, n)
    edge_feat = out_rows.reshape(n, n, _EMB)
    return (edge_feat, x)


# speedup vs baseline: 1.0505x; 1.0001x over previous
import functools

import jax
import jax.numpy as jnp
from jax import lax
from jax.experimental import pallas as pl
from jax.experimental.pallas import tpu as pltpu
from jax.experimental.pallas import tpu_sc as plsc

_EMB = 32
_NW = 32
_CHUNK = 128


def _sc_gather(table, idx):
    b = idx.shape[0]
    d = table.shape[1]
    bpw = b // _NW
    k_chunks = bpw // _CHUNK
    idx2d = idx.reshape(b // _CHUNK, _CHUNK)
    mesh = plsc.VectorSubcoreMesh(core_axis_name="c", subcore_axis_name="s")

    @functools.partial(
        pl.kernel, mesh=mesh,
        out_type=jax.ShapeDtypeStruct((b, d), jnp.float32),
        scratch_types=[pltpu.VMEM((k_chunks, _CHUNK), jnp.int32),
                       pltpu.VMEM((bpw, d), jnp.float32),
                       pltpu.SemaphoreType.DMA],
        compiler_params=pltpu.CompilerParams(use_tc_tiling_on_sc=False),
    )
    def run(table_hbm, idx_hbm, out_hbm, idx_v, rows_v, sem):
        wid = lax.axis_index("s") * 2 + lax.axis_index("c")
        pltpu.sync_copy(idx_hbm.at[pl.ds(wid * k_chunks, k_chunks)], idx_v)
        copies = [pltpu.async_copy(table_hbm.at[idx_v.at[j]],
                                   rows_v.at[pl.ds(j * _CHUNK, _CHUNK)], sem)
                  for j in range(k_chunks)]
        for c in copies:
            c.wait()
        pltpu.sync_copy(rows_v, out_hbm.at[pl.ds(wid * bpw, bpw)])

    return run(table, idx2d)


def _ln(x, g, b):
    m = x.mean(-1, keepdims=True)
    var = ((x - m) ** 2).mean(-1, keepdims=True)
    return (x - m) / jnp.sqrt(var + 1e-6) * g + b


def _mlp(p, x):
    h = jnp.maximum(jnp.dot(x, p["w1"], preferred_element_type=jnp.float32) + p["b1"], 0.0)
    return jnp.dot(h, p["w2"], preferred_element_type=jnp.float32) + p["b2"]


def _attn_body(y, obs, qW, kW, vW, l1g, l1b, w1, b1, w2, b2, l2g, l2b, temp):
    q = jnp.dot(y, qW, preferred_element_type=jnp.float32)
    k = jnp.dot(y, kW, preferred_element_type=jnp.float32)
    vv = jnp.dot(y, vW, preferred_element_type=jnp.float32)
    ok = jnp.dot(obs, kW, preferred_element_type=jnp.float32)
    ov = jnp.dot(obs, vW, preferred_element_type=jnp.float32)
    zo = lax.dot_general(q, ok, (((1,), (1,)), ((), ())),
                         preferred_element_type=jnp.float32) / temp
    zs = jnp.sum(q * k, axis=-1, keepdims=True) / temp
    m = jnp.maximum(jnp.max(zo, axis=-1, keepdims=True), zs)
    es = jnp.exp(zs - m)
    eo = jnp.exp(zo - m)
    denom = es + jnp.sum(eo, axis=-1, keepdims=True)
    v_new = (es * vv + jnp.dot(eo, ov, preferred_element_type=jnp.float32)) / denom
    y = _ln(v_new + y, l1g, l1b)
    h = jnp.maximum(jnp.dot(y, w1, preferred_element_type=jnp.float32) + b1, 0.0)
    h = jnp.dot(h, w2, preferred_element_type=jnp.float32) + b2
    return _ln(h + y, l2g, l2b)


def _attn3_kernel(y_ref, obs_ref, qW, kW, vW, l1g, l1b, w1, b1, w2, b2, l2g, l2b, out_ref):
    y = y_ref[...]
    obs = obs_ref[...]
    temp = jnp.float32(float(_EMB) ** 0.5)
    for blk in range(3):
        y = _attn_body(y, obs, qW[blk], kW[blk], vW[blk], l1g[blk], l1b[blk],
                       w1[blk], b1[blk], w2[blk], b2[blk], l2g[blk], l2b[blk], temp)
    out_ref[...] = y


def _hy_attn3_kernel(d_in, vi_ref, vj_ref, obs_ref, w1h, b1h, w2h, b2h,
                     qW, kW, vW, l1g, l1b, w1, b1, w2, b2, l2g, l2b, out_ref):
    vi = vi_ref[...][:, :d_in]
    vj = vj_ref[...][:, :d_in]
    z = jnp.concatenate([vj - vi, vj, vi], axis=-1)
    h = jnp.maximum(jnp.dot(z, w1h[...], preferred_element_type=jnp.float32) + b1h[...], 0.0)
    y = jnp.dot(h, w2h[...], preferred_element_type=jnp.float32) + b2h[...]
    obs = obs_ref[...]
    temp = jnp.float32(float(_EMB) ** 0.5)
    for blk in range(3):
        y = _attn_body(y, obs, qW[blk], kW[blk], vW[blk], l1g[blk], l1b[blk],
                       w1[blk], b1[blk], w2[blk], b2[blk], l2g[blk], l2b[blk], temp)
    out_ref[...] = y


def _enc_attn3_kernel(z_ref, obs_ref, w1h, b1h, w2h, b2h,
                      qW, kW, vW, l1g, l1b, w1, b1, w2, b2, l2g, l2b, out_ref):
    z = z_ref[...]
    h = jnp.maximum(jnp.dot(z, w1h[...], preferred_element_type=jnp.float32) + b1h[...], 0.0)
    y = jnp.dot(h, w2h[...], preferred_element_type=jnp.float32) + b2h[...]
    obs = obs_ref[...]
    temp = jnp.float32(float(_EMB) ** 0.5)
    for blk in range(3):
        y = _attn_body(y, obs, qW[blk], kW[blk], vW[blk], l1g[blk], l1b[blk],
                       w1[blk], b1[blk], w2[blk], b2[blk], l2g[blk], l2b[blk], temp)
    out_ref[...] = y


def _enc_attn3(z, obs, phx, blocks):
    n = z.shape[0]
    ws = [phx["w1"], phx["b1"], phx["w2"], phx["b2"],
          _stack(blocks, "qW"), _stack(blocks, "kW"), _stack(blocks, "vW"),
          _stack(blocks, "ln1_g"), _stack(blocks, "ln1_b"),
          _stack(blocks, "ff", "w1"), _stack(blocks, "ff", "b1"),
          _stack(blocks, "ff", "w2"), _stack(blocks, "ff", "b2"),
          _stack(blocks, "ln2_g"), _stack(blocks, "ln2_b")]
    full = lambda s: pl.BlockSpec(s, lambda i: (0,) * len(s))
    return pl.pallas_call(
        _enc_attn3_kernel,
        grid=(1,),
        in_specs=[full(z.shape), full(obs.shape)] + [full(w.shape) for w in ws],
        out_specs=full((n, _EMB)),
        out_shape=jax.ShapeDtypeStruct((n, _EMB), jnp.float32),
    )(z, obs, *ws)


def _hy_attn3(g, e, d_in, obs, phy, blocks):
    blk = 4000
    grid = e // blk
    ws = [phy["w1"], phy["b1"], phy["w2"], phy["b2"],
          _stack(blocks, "qW"), _stack(blocks, "kW"), _stack(blocks, "vW"),
          _stack(blocks, "ln1_g"), _stack(blocks, "ln1_b"),
          _stack(blocks, "ff", "w1"), _stack(blocks, "ff", "b1"),
          _stack(blocks, "ff", "w2"), _stack(blocks, "ff", "b2"),
          _stack(blocks, "ln2_g"), _stack(blocks, "ln2_b")]
    full = lambda s: pl.BlockSpec(s, lambda i: (0,) * len(s))
    gspec = lambda off: pl.BlockSpec((blk, g.shape[1]), lambda i, off=off: (i + off, 0))
    return pl.pallas_call(
        functools.partial(_hy_attn3_kernel, d_in),
        grid=(grid,),
        in_specs=[gspec(0), gspec(grid), full(obs.shape)] + [full(w.shape) for w in ws],
        out_specs=pl.BlockSpec((blk, _EMB), lambda i: (i, 0)),
        out_shape=jax.ShapeDtypeStruct((e, _EMB), jnp.float32),
    )(g, g, obs, *ws)


def _edge_mlp_kernel(with_fy, a_ref, b_ref, y_ref, w1y, b1y, w2y, b2y,
                     w1x, b1x, w2x, b2x, yout_ref, msg_ref):
    a = a_ref[...]
    b = b_ref[...]
    y = y_ref[...]
    if with_fy:
        zy = jnp.concatenate([b - a, b, a], axis=-1)
        h = jnp.maximum(jnp.dot(zy, w1y[...], preferred_element_type=jnp.float32) + b1y[...], 0.0)
        y = jnp.maximum(y, jnp.dot(h, w2y[...], preferred_element_type=jnp.float32) + b2y[...])
    yout_ref[...] = y
    zx = jnp.concatenate([a - b, a, b, y], axis=-1)
    h = jnp.maximum(jnp.dot(zx, w1x[...], preferred_element_type=jnp.float32) + b1x[...], 0.0)
    msg_ref[...] = jnp.dot(h, w2x[...], preferred_element_type=jnp.float32) + b2x[...]


def _edge_mlps(gx, y, pfy, pfx, with_fy):
    e = y.shape[0]
    blk = 4000
    grid = e // blk
    full = lambda s: pl.BlockSpec(s, lambda i: (0,) * len(s))
    wargs = [pfy["w1"], pfy["b1"], pfy["w2"], pfy["b2"],
             pfx["w1"], pfx["b1"], pfx["w2"], pfx["b2"]]
    espec = pl.BlockSpec((blk, _EMB), lambda i: (i, 0))
    return pl.pallas_call(
        functools.partial(_edge_mlp_kernel, with_fy),
        grid=(grid,),
        in_specs=[pl.BlockSpec((blk, _EMB), lambda i: (i, 0)),
                  pl.BlockSpec((blk, _EMB), lambda i: (i + grid, 0)),
                  espec] + [full(w.shape) for w in wargs],
        out_specs=(espec, espec),
        out_shape=(jax.ShapeDtypeStruct((e, _EMB), jnp.float32),
                   jax.ShapeDtypeStruct((e, _EMB), jnp.float32)),
    )(gx, gx, y, *wargs)


def _stack(blocks, *path):
    def get(b):
        for k in path:
            b = b[k]
        return b
    return jnp.stack([get(b) for b in blocks])


def _attn3(y, obs, blocks, block_rows):
    n = y.shape[0]
    assert n % block_rows == 0
    grid = n // block_rows
    ws = [_stack(blocks, "qW"), _stack(blocks, "kW"), _stack(blocks, "vW"),
          _stack(blocks, "ln1_g"), _stack(blocks, "ln1_b"),
          _stack(blocks, "ff", "w1"), _stack(blocks, "ff", "b1"),
          _stack(blocks, "ff", "w2"), _stack(blocks, "ff", "b2"),
          _stack(blocks, "ln2_g"), _stack(blocks, "ln2_b")]
    full = lambda s: pl.BlockSpec(s, lambda i: (0,) * len(s))
    in_specs = [pl.BlockSpec((block_rows, _EMB), lambda i: (i, 0)),
                full(obs.shape)] + [full(w.shape) for w in ws]
    return pl.pallas_call(
        _attn3_kernel,
        grid=(grid,),
        in_specs=in_specs,
        out_specs=pl.BlockSpec((block_rows, _EMB), lambda i: (i, 0)),
        out_shape=jax.ShapeDtypeStruct((n, _EMB), jnp.float32),
    )(y, obs, *ws)


_NPAD = 1024
_EPW = 2000
_GPT = 4096


def _sc_mpnn(x_pad, msg, dst, idx2d):
    mesh = plsc.VectorSubcoreMesh(core_axis_name="c", subcore_axis_name="s",
                                  num_cores=1)

    @functools.partial(
        pl.kernel, mesh=mesh,
        out_type=(jax.ShapeDtypeStruct((_NPAD, 32), jnp.float32),
                  jax.ShapeDtypeStruct((16 * _GPT, 32), jnp.float32),
                  jax.ShapeDtypeStruct((16, _NPAD, 32), jnp.float32)),
        scratch_types=[pltpu.VMEM((2048, 32), jnp.float32),
                       pltpu.VMEM((_EPW,), jnp.int32),
                       pltpu.VMEM((_NPAD, 32), jnp.float32),
                       pltpu.VMEM((64, 32), jnp.float32),
                       pltpu.VMEM((64, 32), jnp.float32),
                       pltpu.VMEM((32, 128), jnp.int32),
                       pltpu.SemaphoreType.DMA],
        compiler_params=pltpu.CompilerParams(use_tc_tiling_on_sc=False),
    )
    def run(x_hbm, msg_hbm, dst_hbm, idx_hbm, xout_hbm, gx_hbm, part_hbm,
            buf_v, dst_v, acc_v, comb_v, tmp_v, idx_v, sem):
        t = lax.axis_index("s")
        pltpu.sync_copy(x_hbm, acc_v)
        pltpu.sync_copy(msg_hbm.at[pl.ds(t * _EPW, _EPW)],
                        buf_v.at[pl.ds(0, _EPW)])
        pltpu.sync_copy(dst_hbm.at[pl.ds(t * _EPW, _EPW)], dst_v)

        def edge_body(g, carry):
            dvec = dst_v[pl.ds(g * 16, 16)]
            base = g * 16
            for l in range(16):
                d = dvec[l]
                for h in (0, 16):
                    m = buf_v[base + l, pl.ds(h, 16)]
                    a = acc_v[d, pl.ds(h, 16)]
                    acc_v[d, pl.ds(h, 16)] = jnp.maximum(a, m)
            return carry

        lax.fori_loop(0, _EPW // 16, edge_body, 0)
        pltpu.sync_copy(acc_v, part_hbm.at[t])
        plsc.subcore_barrier()
        pltpu.sync_copy(part_hbm.at[0, pl.ds(t * 64, 64)], comb_v)

        def comb_body(j, carry):
            pltpu.sync_copy(part_hbm.at[j, pl.ds(t * 64, 64)], tmp_v)

            def row_body(r, c2):
                for h in (0, 16):
                    comb_v[r, pl.ds(h, 16)] = jnp.maximum(
                        comb_v[r, pl.ds(h, 16)], tmp_v[r, pl.ds(h, 16)])
                return c2

            return lax.fori_loop(0, 64, row_body, carry)

        lax.fori_loop(1, 16, comb_body, 0)
        pltpu.sync_copy(comb_v, xout_hbm.at[pl.ds(t * 64, 64)])
        plsc.subcore_barrier()
        pltpu.sync_copy(idx_hbm.at[pl.ds(t * 32, 32)], idx_v)
        for h in range(2):
            copies = [pltpu.async_copy(xout_hbm.at[idx_v.at[h * 16 + j]],
                                       buf_v.at[pl.ds(j * 128, 128)], sem)
                      for j in range(16)]
            for cp in copies:
                cp.wait()
            pltpu.sync_copy(buf_v, gx_hbm.at[pl.ds(t * _GPT + h * 2048, 2048)])

    return run(x_pad, msg, dst, idx2d)


_TROWS = 31250
_EPT = 2048
_ZCH = 1536


def _sc_assemble(y_pad, idx2, n):
    mesh = plsc.VectorSubcoreMesh(core_axis_name="c", subcore_axis_name="s")
    zsrc = jnp.zeros((_ZCH, 32), jnp.float32)

    @functools.partial(
        pl.kernel, mesh=mesh,
        out_type=jax.ShapeDtypeStruct((n * n, 32), jnp.float32),
        scratch_types=[pltpu.VMEM((_ZCH, 32), jnp.float32),
                       pltpu.VMEM((_EPT, 32), jnp.float32),
                       pltpu.VMEM((16, 128), jnp.int32),
                       pltpu.SemaphoreType.DMA,
                       pltpu.SemaphoreType.DMA],
        compiler_params=pltpu.CompilerParams(use_tc_tiling_on_sc=False),
    )
    def run(y_hbm, idx_hbm, zsrc_hbm, out_hbm, zbuf, rows_v, idx_v, zsem, ssem):
        c = lax.axis_index("c")
        t = lax.axis_index("s")
        pltpu.sync_copy(zsrc_hbm, zbuf)
        base = (c * 16 + t) * _TROWS
        zcopies = [pltpu.async_copy(zbuf, out_hbm.at[pl.ds(base + k * _ZCH, _ZCH)], zsem)
                   for k in range(20)]
        zcopies.append(pltpu.async_copy(zbuf.at[pl.ds(0, 530)],
                                        out_hbm.at[pl.ds(base + 20 * _ZCH, 530)], zsem))
        pltpu.sync_copy(y_hbm.at[pl.ds(t * _EPT, _EPT)], rows_v)
        pltpu.sync_copy(idx_hbm.at[pl.ds(t * 16, 16)], idx_v)
        for cp in zcopies:
            cp.wait()
        plsc.subcore_barrier()
        scopies = [pltpu.async_copy(rows_v.at[pl.ds(j * 128, 128)],
                                    out_hbm.at[idx_v.at[j]], ssem)
                   for j in range(16)]
        for cp in scopies:
            cp.wait()

    return run(y_pad, idx2, zsrc)


def kernel(v, labels, obstacles, pos_enc, edge_index, loop, params):
    n = v.shape[0]
    vcat = jnp.concatenate([v, labels], axis=-1)
    goal_idx = jnp.argmin(jnp.abs(labels[:, 0] - 1.0))
    goal = vcat[goal_idx][None, :]
    gr = jnp.broadcast_to(goal, vcat.shape)
    zx0 = jnp.concatenate([vcat, gr, vcat - gr, (vcat - gr) ** 2], axis=-1)

    src = edge_index[0]
    dst = edge_index[1]
    e = src.shape[0]
    both = jnp.concatenate([src, dst]).astype(jnp.int32)
    padlen = (-both.shape[0]) % (_NW * _CHUNK)
    both_pad = jnp.concatenate(
        [both, jnp.arange(padlen, dtype=jnp.int32) % n])

    vcat16 = jnp.pad(vcat, ((0, 0), (0, 16 - vcat.shape[1])))
    g = _sc_gather(vcat16, both_pad)

    obs_node = _mlp(params["onc"], obstacles) + pos_enc
    obs_edge = _mlp(params["oec"], obstacles) + pos_enc

    x = _enc_attn3(zx0, obs_node, params["hx"], params["na"])
    y = _hy_attn3(g, e, vcat.shape[1], obs_edge, params["hy"], params["ea"])

    dst32 = dst.astype(jnp.int32)
    idx2d = both_pad.reshape(512, 128)
    x_pad = jnp.concatenate([x, jnp.zeros((_NPAD - n, _EMB), jnp.float32)])
    gx0 = _sc_gather(x, both_pad)
    _, msg0 = _edge_mlps(gx0, y, params["fy"], params["fx"], with_fy=False)

    def body(_, carry):
        x_pad, y, msg = carry
        x_pad, gx, _ = _sc_mpnn(x_pad, msg, dst32, idx2d)
        y, msg = _edge_mlps(gx, y, params["fy"], params["fx"], with_fy=True)
        return (x_pad, y, msg)

    x_pad, y, _ = lax.fori_loop(0, loop, body, (x_pad, y, msg0))
    x = x_pad[:n]

    e_pad = 16 * _EPT
    y_pad = jnp.concatenate(
        [y, jnp.broadcast_to(y[0], (e_pad - e, _EMB))])
    flat = src.astype(jnp.int32) * n + dst.astype(jnp.int32)
    flat_p = jnp.concatenate(
        [flat, jnp.broadcast_to(flat[0], (e_pad - e,))])
    idx2 = flat_p.reshape(256, 128)
    out_rows = _sc_assemble(y_pad, idx2, n)
    edge_feat = out_rows.reshape(n, n, _EMB)
    return (edge_feat, x)


# speedup vs baseline: 1.0518x; 1.0012x over previous
import functools

import jax
import jax.numpy as jnp
from jax import lax
from jax.experimental import pallas as pl
from jax.experimental.pallas import tpu as pltpu
from jax.experimental.pallas import tpu_sc as plsc

_EMB = 32
_NW = 32
_CHUNK = 128


def _sc_gather(table, idx):
    b = idx.shape[0]
    d = table.shape[1]
    bpw = b // _NW
    k_chunks = bpw // _CHUNK
    idx2d = idx.reshape(b // _CHUNK, _CHUNK)
    mesh = plsc.VectorSubcoreMesh(core_axis_name="c", subcore_axis_name="s")

    @functools.partial(
        pl.kernel, mesh=mesh,
        out_type=jax.ShapeDtypeStruct((b, d), jnp.float32),
        scratch_types=[pltpu.VMEM((k_chunks, _CHUNK), jnp.int32),
                       pltpu.VMEM((bpw, d), jnp.float32),
                       pltpu.SemaphoreType.DMA],
        compiler_params=pltpu.CompilerParams(use_tc_tiling_on_sc=False),
    )
    def run(table_hbm, idx_hbm, out_hbm, idx_v, rows_v, sem):
        wid = lax.axis_index("s") * 2 + lax.axis_index("c")
        pltpu.sync_copy(idx_hbm.at[pl.ds(wid * k_chunks, k_chunks)], idx_v)
        copies = [pltpu.async_copy(table_hbm.at[idx_v.at[j]],
                                   rows_v.at[pl.ds(j * _CHUNK, _CHUNK)], sem)
                  for j in range(k_chunks)]
        for c in copies:
            c.wait()
        pltpu.sync_copy(rows_v, out_hbm.at[pl.ds(wid * bpw, bpw)])

    return run(table, idx2d)


def _ln(x, g, b):
    m = x.mean(-1, keepdims=True)
    var = ((x - m) ** 2).mean(-1, keepdims=True)
    return (x - m) / jnp.sqrt(var + 1e-6) * g + b


def _mlp(p, x):
    h = jnp.maximum(jnp.dot(x, p["w1"], preferred_element_type=jnp.float32) + p["b1"], 0.0)
    return jnp.dot(h, p["w2"], preferred_element_type=jnp.float32) + p["b2"]


def _attn_body(y, obs, qW, kW, vW, l1g, l1b, w1, b1, w2, b2, l2g, l2b, temp):
    q = jnp.dot(y, qW, preferred_element_type=jnp.float32)
    k = jnp.dot(y, kW, preferred_element_type=jnp.float32)
    vv = jnp.dot(y, vW, preferred_element_type=jnp.float32)
    ok = jnp.dot(obs, kW, preferred_element_type=jnp.float32)
    ov = jnp.dot(obs, vW, preferred_element_type=jnp.float32)
    zo = lax.dot_general(q, ok, (((1,), (1,)), ((), ())),
                         preferred_element_type=jnp.float32) / temp
    zs = jnp.sum(q * k, axis=-1, keepdims=True) / temp
    m = jnp.maximum(jnp.max(zo, axis=-1, keepdims=True), zs)
    es = jnp.exp(zs - m)
    eo = jnp.exp(zo - m)
    denom = es + jnp.sum(eo, axis=-1, keepdims=True)
    v_new = (es * vv + jnp.dot(eo, ov, preferred_element_type=jnp.float32)) / denom
    y = _ln(v_new + y, l1g, l1b)
    h = jnp.maximum(jnp.dot(y, w1, preferred_element_type=jnp.float32) + b1, 0.0)
    h = jnp.dot(h, w2, preferred_element_type=jnp.float32) + b2
    return _ln(h + y, l2g, l2b)


def _hy_attn3_kernel(d_in, vi_ref, vj_ref, obs_ref, w1h, b1h, w2h, b2h,
                     qW, kW, vW, l1g, l1b, w1, b1, w2, b2, l2g, l2b, out_ref):
    vi = vi_ref[...][:, :d_in]
    vj = vj_ref[...][:, :d_in]
    z = jnp.concatenate([vj - vi, vj, vi], axis=-1)
    h = jnp.maximum(jnp.dot(z, w1h[...], preferred_element_type=jnp.float32) + b1h[...], 0.0)
    y = jnp.dot(h, w2h[...], preferred_element_type=jnp.float32) + b2h[...]
    obs = obs_ref[...]
    temp = jnp.float32(float(_EMB) ** 0.5)
    for blk in range(3):
        y = _attn_body(y, obs, qW[blk], kW[blk], vW[blk], l1g[blk], l1b[blk],
                       w1[blk], b1[blk], w2[blk], b2[blk], l2g[blk], l2b[blk], temp)
    out_ref[...] = y


def _enc_attn3_kernel(z_ref, obs_ref, w1h, b1h, w2h, b2h,
                      qW, kW, vW, l1g, l1b, w1, b1, w2, b2, l2g, l2b, out_ref):
    z = z_ref[...]
    h = jnp.maximum(jnp.dot(z, w1h[...], preferred_element_type=jnp.float32) + b1h[...], 0.0)
    y = jnp.dot(h, w2h[...], preferred_element_type=jnp.float32) + b2h[...]
    obs = obs_ref[...]
    temp = jnp.float32(float(_EMB) ** 0.5)
    for blk in range(3):
        y = _attn_body(y, obs, qW[blk], kW[blk], vW[blk], l1g[blk], l1b[blk],
                       w1[blk], b1[blk], w2[blk], b2[blk], l2g[blk], l2b[blk], temp)
    out_ref[...] = y


def _enc_attn3(z, obs, phx, blocks):
    n = z.shape[0]
    ws = [phx["w1"], phx["b1"], phx["w2"], phx["b2"],
          _stack(blocks, "qW"), _stack(blocks, "kW"), _stack(blocks, "vW"),
          _stack(blocks, "ln1_g"), _stack(blocks, "ln1_b"),
          _stack(blocks, "ff", "w1"), _stack(blocks, "ff", "b1"),
          _stack(blocks, "ff", "w2"), _stack(blocks, "ff", "b2"),
          _stack(blocks, "ln2_g"), _stack(blocks, "ln2_b")]
    full = lambda s: pl.BlockSpec(s, lambda i: (0,) * len(s))
    return pl.pallas_call(
        _enc_attn3_kernel,
        grid=(1,),
        in_specs=[full(z.shape), full(obs.shape)] + [full(w.shape) for w in ws],
        out_specs=full((n, _EMB)),
        out_shape=jax.ShapeDtypeStruct((n, _EMB), jnp.float32),
    )(z, obs, *ws)


def _hy_attn3(g, e, d_in, obs, phy, blocks):
    blk = 4000
    grid = e // blk
    ws = [phy["w1"], phy["b1"], phy["w2"], phy["b2"],
          _stack(blocks, "qW"), _stack(blocks, "kW"), _stack(blocks, "vW"),
          _stack(blocks, "ln1_g"), _stack(blocks, "ln1_b"),
          _stack(blocks, "ff", "w1"), _stack(blocks, "ff", "b1"),
          _stack(blocks, "ff", "w2"), _stack(blocks, "ff", "b2"),
          _stack(blocks, "ln2_g"), _stack(blocks, "ln2_b")]
    full = lambda s: pl.BlockSpec(s, lambda i: (0,) * len(s))
    gspec = lambda off: pl.BlockSpec((blk, g.shape[1]), lambda i, off=off: (i + off, 0))
    return pl.pallas_call(
        functools.partial(_hy_attn3_kernel, d_in),
        grid=(grid,),
        in_specs=[gspec(0), gspec(grid), full(obs.shape)] + [full(w.shape) for w in ws],
        out_specs=pl.BlockSpec((blk, _EMB), lambda i: (i, 0)),
        out_shape=jax.ShapeDtypeStruct((e, _EMB), jnp.float32),
    )(g, g, obs, *ws)


def _edge_mlp_kernel(with_fy, a_ref, b_ref, y_ref, w1y, b1y, w2y, b2y,
                     w1x, b1x, w2x, b2x, yout_ref, msg_ref):
    a = a_ref[...]
    b = b_ref[...]
    y = y_ref[...]
    if with_fy:
        zy = jnp.concatenate([b - a, b, a], axis=-1)
        h = jnp.maximum(jnp.dot(zy, w1y[...], preferred_element_type=jnp.float32) + b1y[...], 0.0)
        y = jnp.maximum(y, jnp.dot(h, w2y[...], preferred_element_type=jnp.float32) + b2y[...])
    yout_ref[...] = y
    zx = jnp.concatenate([a - b, a, b, y], axis=-1)
    h = jnp.maximum(jnp.dot(zx, w1x[...], preferred_element_type=jnp.float32) + b1x[...], 0.0)
    msg_ref[...] = jnp.dot(h, w2x[...], preferred_element_type=jnp.float32) + b2x[...]


def _edge_mlps(gx, y, pfy, pfx, with_fy):
    e = y.shape[0]
    blk = 4000
    grid = e // blk
    full = lambda s: pl.BlockSpec(s, lambda i: (0,) * len(s))
    wargs = [pfy["w1"], pfy["b1"], pfy["w2"], pfy["b2"],
             pfx["w1"], pfx["b1"], pfx["w2"], pfx["b2"]]
    espec = pl.BlockSpec((blk, _EMB), lambda i: (i, 0))
    return pl.pallas_call(
        functools.partial(_edge_mlp_kernel, with_fy),
        grid=(grid,),
        in_specs=[pl.BlockSpec((blk, _EMB), lambda i: (i, 0)),
                  pl.BlockSpec((blk, _EMB), lambda i: (i + grid, 0)),
                  espec] + [full(w.shape) for w in wargs],
        out_specs=(espec, espec),
        out_shape=(jax.ShapeDtypeStruct((e, _EMB), jnp.float32),
                   jax.ShapeDtypeStruct((e, _EMB), jnp.float32)),
    )(gx, gx, y, *wargs)


def _stack(blocks, *path):
    def get(b):
        for k in path:
            b = b[k]
        return b
    return jnp.stack([get(b) for b in blocks])


_NPAD = 1024
_EPW = 2000
_GPT = 4096


def _sc_mpnn(x_pad, msg, dst, idx2d):
    mesh = plsc.VectorSubcoreMesh(core_axis_name="c", subcore_axis_name="s",
                                  num_cores=1)

    @functools.partial(
        pl.kernel, mesh=mesh,
        out_type=(jax.ShapeDtypeStruct((_NPAD, 32), jnp.float32),
                  jax.ShapeDtypeStruct((16 * _GPT, 32), jnp.float32),
                  jax.ShapeDtypeStruct((16, _NPAD, 32), jnp.float32)),
        scratch_types=[pltpu.VMEM((2048, 32), jnp.float32),
                       pltpu.VMEM((_EPW,), jnp.int32),
                       pltpu.VMEM((_NPAD, 32), jnp.float32),
                       pltpu.VMEM((64, 32), jnp.float32),
                       pltpu.VMEM((64, 32), jnp.float32),
                       pltpu.VMEM((32, 128), jnp.int32),
                       pltpu.SemaphoreType.DMA],
        compiler_params=pltpu.CompilerParams(use_tc_tiling_on_sc=False),
    )
    def run(x_hbm, msg_hbm, dst_hbm, idx_hbm, xout_hbm, gx_hbm, part_hbm,
            buf_v, dst_v, acc_v, comb_v, tmp_v, idx_v, sem):
        t = lax.axis_index("s")
        pltpu.sync_copy(x_hbm, acc_v)
        pltpu.sync_copy(msg_hbm.at[pl.ds(t * _EPW, _EPW)],
                        buf_v.at[pl.ds(0, _EPW)])
        pltpu.sync_copy(dst_hbm.at[pl.ds(t * _EPW, _EPW)], dst_v)

        def edge_body(g, carry):
            dvec = dst_v[pl.ds(g * 16, 16)]
            base = g * 16
            for l in range(16):
                d = dvec[l]
                for h in (0, 16):
                    m = buf_v[base + l, pl.ds(h, 16)]
                    a = acc_v[d, pl.ds(h, 16)]
                    acc_v[d, pl.ds(h, 16)] = jnp.maximum(a, m)
            return carry

        lax.fori_loop(0, _EPW // 16, edge_body, 0)
        pltpu.sync_copy(acc_v, part_hbm.at[t])
        plsc.subcore_barrier()
        pltpu.sync_copy(part_hbm.at[0, pl.ds(t * 64, 64)], comb_v)

        def comb_body(j, carry):
            pltpu.sync_copy(part_hbm.at[j, pl.ds(t * 64, 64)], tmp_v)

            def row_body(r, c2):
                for h in (0, 16):
                    comb_v[r, pl.ds(h, 16)] = jnp.maximum(
                        comb_v[r, pl.ds(h, 16)], tmp_v[r, pl.ds(h, 16)])
                return c2

            return lax.fori_loop(0, 64, row_body, carry)

        lax.fori_loop(1, 16, comb_body, 0)
        pltpu.sync_copy(comb_v, xout_hbm.at[pl.ds(t * 64, 64)])
        plsc.subcore_barrier()
        pltpu.sync_copy(idx_hbm.at[pl.ds(t * 32, 32)], idx_v)
        for h in range(2):
            copies = [pltpu.async_copy(xout_hbm.at[idx_v.at[h * 16 + j]],
                                       buf_v.at[pl.ds(j * 128, 128)], sem)
                      for j in range(16)]
            for cp in copies:
                cp.wait()
            pltpu.sync_copy(buf_v, gx_hbm.at[pl.ds(t * _GPT + h * 2048, 2048)])

    return run(x_pad, msg, dst, idx2d)


_TROWS = 31250
_EPT = 2048
_ZCH = 1536


def _sc_assemble(y_pad, idx2, n):
    mesh = plsc.VectorSubcoreMesh(core_axis_name="c", subcore_axis_name="s")
    zsrc = jnp.zeros((_ZCH, 32), jnp.float32)

    @functools.partial(
        pl.kernel, mesh=mesh,
        out_type=jax.ShapeDtypeStruct((n * n, 32), jnp.float32),
        scratch_types=[pltpu.VMEM((_ZCH, 32), jnp.float32),
                       pltpu.VMEM((_EPT, 32), jnp.float32),
                       pltpu.VMEM((16, 128), jnp.int32),
                       pltpu.SemaphoreType.DMA,
                       pltpu.SemaphoreType.DMA],
        compiler_params=pltpu.CompilerParams(use_tc_tiling_on_sc=False),
    )
    def run(y_hbm, idx_hbm, zsrc_hbm, out_hbm, zbuf, rows_v, idx_v, zsem, ssem):
        c = lax.axis_index("c")
        t = lax.axis_index("s")
        pltpu.sync_copy(zsrc_hbm, zbuf)
        base = (c * 16 + t) * _TROWS
        zcopies = [pltpu.async_copy(zbuf, out_hbm.at[pl.ds(base + k * _ZCH, _ZCH)], zsem)
                   for k in range(20)]
        zcopies.append(pltpu.async_copy(zbuf.at[pl.ds(0, 530)],
                                        out_hbm.at[pl.ds(base + 20 * _ZCH, 530)], zsem))
        pltpu.sync_copy(y_hbm.at[pl.ds(t * _EPT, _EPT)], rows_v)
        pltpu.sync_copy(idx_hbm.at[pl.ds(t * 16, 16)], idx_v)
        for cp in zcopies:
            cp.wait()
        plsc.subcore_barrier()
        scopies = [pltpu.async_copy(rows_v.at[pl.ds(j * 128, 128)],
                                    out_hbm.at[idx_v.at[j]], ssem)
                   for j in range(16)]
        for cp in scopies:
            cp.wait()

    return run(y_pad, idx2, zsrc)


def kernel(v, labels, obstacles, pos_enc, edge_index, loop, params):
    n = v.shape[0]
    vcat = jnp.concatenate([v, labels], axis=-1)
    goal_idx = jnp.argmin(jnp.abs(labels[:, 0] - 1.0))
    goal = vcat[goal_idx][None, :]
    gr = jnp.broadcast_to(goal, vcat.shape)
    zx0 = jnp.concatenate([vcat, gr, vcat - gr, (vcat - gr) ** 2], axis=-1)

    src = edge_index[0]
    dst = edge_index[1]
    e = src.shape[0]
    both = jnp.concatenate([src, dst]).astype(jnp.int32)
    padlen = (-both.shape[0]) % (_NW * _CHUNK)
    both_pad = jnp.concatenate(
        [both, jnp.arange(padlen, dtype=jnp.int32) % n])

    vcat16 = jnp.pad(vcat, ((0, 0), (0, 16 - vcat.shape[1])))
    g = _sc_gather(vcat16, both_pad)

    obs_node = _mlp(params["onc"], obstacles) + pos_enc
    obs_edge = _mlp(params["oec"], obstacles) + pos_enc

    x = _enc_attn3(zx0, obs_node, params["hx"], params["na"])
    y = _hy_attn3(g, e, vcat.shape[1], obs_edge, params["hy"], params["ea"])

    dst32 = dst.astype(jnp.int32)
    idx2d = both_pad.reshape(512, 128)
    x_pad = jnp.concatenate([x, jnp.zeros((_NPAD - n, _EMB), jnp.float32)])
    gx0 = _sc_gather(x, both_pad)
    _, msg0 = _edge_mlps(gx0, y, params["fy"], params["fx"], with_fy=False)

    def body(_, carry):
        x_pad, y, msg = carry
        x_pad, gx, _ = _sc_mpnn(x_pad, msg, dst32, idx2d)
        y, msg = _edge_mlps(gx, y, params["fy"], params["fx"], with_fy=True)
        return (x_pad, y, msg)

    x_pad, y, _ = lax.fori_loop(0, loop, body, (x_pad, y, msg0))
    x = x_pad[:n]

    e_pad = 16 * _EPT
    y_pad = jnp.concatenate(
        [y, jnp.broadcast_to(y[0], (e_pad - e, _EMB))])
    flat = src.astype(jnp.int32) * n + dst.astype(jnp.int32)
    flat_p = jnp.concatenate(
        [flat, jnp.broadcast_to(flat[0], (e_pad - e,))])
    idx2 = flat_p.reshape(256, 128)
    out_rows = _sc_assemble(y_pad, idx2, n)
    edge_feat = out_rows.reshape(n, n, _EMB)
    return (edge_feat, x)
